# Initial kernel scaffold; baseline (speedup 1.0000x reference)
import jax
import jax.numpy as jnp
from jax.experimental import pallas as pl


def kernel(x, edge_index, W1, b1, W2, b2, Wlin, blin):
    raise NotImplementedError("write your pallas kernel here")


# speedup vs baseline: 3.8004x; 3.8004x over previous
import functools

import jax
import jax.numpy as jnp
from jax import lax
from jax.experimental import pallas as pl
from jax.experimental.pallas import tpu as pltpu
from jax.experimental.pallas import tpu_sc as plsc

_N = 50
_NP = 64
_F = 5
_E = 800
_L = 16
_CH = _E // _L
_TBL = 1024

_W1_OFF = 0
_W2_OFF = 25
_WL_OFF = 50
_B1_OFF = 55
_B2_OFF = 60
_BL_OFF = 65
_NPAR = 66


def _sc_body(ei_hbm, xcm_hbm, par_hbm, tbl_hbm, out_hbm,
             src_v, dst_v, x_v, h_v, agg_v, par_v, tbl_v, deg_v, dinv_v, o_v):
    is_t0 = (lax.axis_index("c") == 0) & (lax.axis_index("s") == 0)

    @pl.when(is_t0)
    def _():
        pltpu.sync_copy(ei_hbm.at[0], src_v)
        pltpu.sync_copy(ei_hbm.at[1], dst_v)
        pltpu.sync_copy(xcm_hbm, x_v)
        pltpu.sync_copy(par_hbm, par_v)
        pltpu.sync_copy(tbl_hbm, tbl_v)

        zeros = jnp.zeros((_L,), jnp.float32)
        ones = jnp.ones((_L,), jnp.float32)

        for i in range(_NP // _L):
            deg_v[pl.ds(i * _L, _L)] = ones

        def deg_step(c, _):
            d_idx = dst_v[pl.ds(c * _L, _L)]
            plsc.addupdate_scatter(deg_v, [d_idx], ones)
            return 0
        lax.fori_loop(0, _CH, deg_step, 0)

        for i in range(_NP // _L):
            sl = pl.ds(i * _L, _L)
            di = deg_v[sl].astype(jnp.int32)
            dinv_v[sl] = plsc.load_gather(tbl_v, [di])

        def par(r):
            return par_v[pl.ds(r * _L, _L)]

        for layer in range(2):
            w_off = _W1_OFF if layer == 0 else _W2_OFF
            b_off = _B1_OFF if layer == 0 else _B2_OFF
            src_feat = x_v if layer == 0 else h_v

            for i in range(_F * _NP // _L):
                agg_v[pl.ds(i * _L, _L)] = zeros

            def edge_step(c, _):
                base = c * _L
                s_idx = src_v[pl.ds(base, _L)]
                d_idx = dst_v[pl.ds(base, _L)]
                norm = (plsc.load_gather(dinv_v, [s_idx])
                        * plsc.load_gather(dinv_v, [d_idx]))
                for j in range(_F):
                    off = j * _NP
                    vals = plsc.load_gather(src_feat, [s_idx + off]) * norm
                    plsc.addupdate_scatter(agg_v, [d_idx + off], vals)
                return 0
            lax.fori_loop(0, _CH, edge_step, 0)

            for i in range(_NP // _L):
                dv = dinv_v[pl.ds(i * _L, _L)]
                d2 = dv * dv
                for j in range(_F):
                    sl = pl.ds(j * _NP + i * _L, _L)
                    agg_v[sl] = agg_v[sl] + d2 * src_feat[sl]

            for j in range(_F):
                wcol = [par(w_off + k * _F + j) for k in range(_F)]
                bj = par(b_off + j)
                for i in range(_NP // _L):
                    acc = bj
                    for k in range(_F):
                        acc = acc + wcol[k] * agg_v[pl.ds(k * _NP + i * _L, _L)]
                    h_v[pl.ds(j * _NP + i * _L, _L)] = jnp.maximum(acc, 0.0)

        wl = [par(_WL_OFF + k) for k in range(_F)]
        bl = par(_BL_OFF)
        for i in range(_NP // _L):
            acc = bl
            for k in range(_F):
                acc = acc + wl[k] * h_v[pl.ds(k * _NP + i * _L, _L)]
            o_v[pl.ds(i * _L, _L)] = acc

        pltpu.sync_copy(o_v, out_hbm)


@jax.jit
def _run(ei, xcm, params, tbl):
    mesh = plsc.VectorSubcoreMesh(core_axis_name="c", subcore_axis_name="s")
    f = pl.kernel(
        _sc_body,
        out_type=jax.ShapeDtypeStruct((_NP,), jnp.float32),
        mesh=mesh,
        compiler_params=pltpu.CompilerParams(needs_layout_passes=False),
        scratch_types=[
            pltpu.VMEM((_E,), jnp.int32),
            pltpu.VMEM((_E,), jnp.int32),
            pltpu.VMEM((_F * _NP,), jnp.float32),
            pltpu.VMEM((_F * _NP,), jnp.float32),
            pltpu.VMEM((_F * _NP,), jnp.float32),
            pltpu.VMEM((_NPAR * _L,), jnp.float32),
            pltpu.VMEM((_TBL,), jnp.float32),
            pltpu.VMEM((_NP,), jnp.float32),
            pltpu.VMEM((_NP,), jnp.float32),
            pltpu.VMEM((_NP,), jnp.float32),
        ],
    )
    return f(ei, xcm, params, tbl)


def kernel(x, edge_index, W1, b1, W2, b2, Wlin, blin):
    xcm = jnp.zeros((_F, _NP), jnp.float32).at[:, :_N].set(x.T).reshape(_F * _NP)

    def splat(v):
        return jnp.broadcast_to(v.reshape(-1, 1), (v.size, _L))

    params = jnp.concatenate(
        [splat(W1.reshape(-1)), splat(W2.reshape(-1)), splat(Wlin.reshape(-1)),
         splat(b1), splat(b2), splat(blin)], axis=0).reshape(_NPAR * _L)

    ar = jnp.arange(_TBL, dtype=jnp.float32)
    tbl = jnp.where(ar > 0, ar ** -0.5, 0.0)

    out = _run(edge_index.astype(jnp.int32), xcm, params, tbl)
    return out[:_N].reshape(_N, 1)


# speedup vs baseline: 3.9877x; 1.0493x over previous
import functools

import jax
import jax.numpy as jnp
from jax import lax
from jax.experimental import pallas as pl
from jax.experimental.pallas import tpu as pltpu
from jax.experimental.pallas import tpu_sc as plsc

_N = 50
_NP = 64
_F = 5
_E = 800
_L = 16
_CH = _E // _L
_TBL = 1024

_W1_OFF = 0
_W2_OFF = 25
_WL_OFF = 50
_B1_OFF = 55
_B2_OFF = 60
_BL_OFF = 65
_NPAR = 66


def _sc_body(ei_hbm, xcm_hbm, par_hbm, tbl_hbm, out_hbm,
             src_v, dst_v, x_v, h_v, agg_v, par_v, tbl_v, deg_v, dinv_v, o_v):
    is_t0 = (lax.axis_index("c") == 0) & (lax.axis_index("s") == 0)

    @pl.when(is_t0)
    def _():
        pltpu.sync_copy(ei_hbm.at[0], src_v)
        pltpu.sync_copy(ei_hbm.at[1], dst_v)
        pltpu.sync_copy(xcm_hbm, x_v)
        pltpu.sync_copy(par_hbm, par_v)
        pltpu.sync_copy(tbl_hbm, tbl_v)

        zeros = jnp.zeros((_L,), jnp.float32)
        ones = jnp.ones((_L,), jnp.float32)

        for i in range(_NP // _L):
            deg_v[pl.ds(i * _L, _L)] = ones

        def deg_step(c, _):
            d_idx = dst_v[pl.ds(c * _L, _L)]
            plsc.addupdate_scatter(deg_v, [d_idx], ones)
            return 0
        lax.fori_loop(0, _CH, deg_step, 0)

        for i in range(_NP // _L):
            sl = pl.ds(i * _L, _L)
            di = deg_v[sl].astype(jnp.int32)
            dinv_v[sl] = plsc.load_gather(tbl_v, [di])

        def par(r):
            return par_v[pl.ds(r * _L, _L)]

        for layer in range(2):
            w_off = _W1_OFF if layer == 0 else _W2_OFF
            b_off = _B1_OFF if layer == 0 else _B2_OFF
            src_feat = x_v if layer == 0 else h_v

            for i in range(_F * _NP // _L):
                agg_v[pl.ds(i * _L, _L)] = zeros

            def edge_step(c, _):
                base = c * _L
                s_idx = src_v[pl.ds(base, _L)]
                d_idx = dst_v[pl.ds(base, _L)]
                norm = (plsc.load_gather(dinv_v, [s_idx])
                        * plsc.load_gather(dinv_v, [d_idx]))
                for j in range(_F):
                    off = j * _NP
                    vals = plsc.load_gather(src_feat, [s_idx + off]) * norm
                    plsc.addupdate_scatter(agg_v, [d_idx + off], vals)
                return 0
            lax.fori_loop(0, _CH, edge_step, 0)

            for i in range(_NP // _L):
                dv = dinv_v[pl.ds(i * _L, _L)]
                d2 = dv * dv
                for j in range(_F):
                    sl = pl.ds(j * _NP + i * _L, _L)
                    agg_v[sl] = agg_v[sl] + d2 * src_feat[sl]

            for j in range(_F):
                wcol = [par(w_off + k * _F + j) for k in range(_F)]
                bj = par(b_off + j)
                for i in range(_NP // _L):
                    acc = bj
                    for k in range(_F):
                        acc = acc + wcol[k] * agg_v[pl.ds(k * _NP + i * _L, _L)]
                    h_v[pl.ds(j * _NP + i * _L, _L)] = jnp.maximum(acc, 0.0)

        wl = [par(_WL_OFF + k) for k in range(_F)]
        bl = par(_BL_OFF)
        for i in range(_NP // _L):
            acc = bl
            for k in range(_F):
                acc = acc + wl[k] * h_v[pl.ds(k * _NP + i * _L, _L)]
            o_v[pl.ds(i * _L, _L)] = acc

        pltpu.sync_copy(o_v, out_hbm)


@jax.jit
def _run(ei, xcm, params, tbl):
    mesh = plsc.VectorSubcoreMesh(core_axis_name="c", subcore_axis_name="s",
                                  num_cores=1)
    f = pl.kernel(
        _sc_body,
        out_type=jax.ShapeDtypeStruct((_NP,), jnp.float32),
        mesh=mesh,
        compiler_params=pltpu.CompilerParams(needs_layout_passes=False),
        scratch_types=[
            pltpu.VMEM((_E,), jnp.int32),
            pltpu.VMEM((_E,), jnp.int32),
            pltpu.VMEM((_F * _NP,), jnp.float32),
            pltpu.VMEM((_F * _NP,), jnp.float32),
            pltpu.VMEM((_F * _NP,), jnp.float32),
            pltpu.VMEM((_NPAR * _L,), jnp.float32),
            pltpu.VMEM((_TBL,), jnp.float32),
            pltpu.VMEM((_NP,), jnp.float32),
            pltpu.VMEM((_NP,), jnp.float32),
            pltpu.VMEM((_NP,), jnp.float32),
        ],
    )
    return f(ei, xcm, params, tbl)


def kernel(x, edge_index, W1, b1, W2, b2, Wlin, blin):
    xcm = jnp.zeros((_F, _NP), jnp.float32).at[:, :_N].set(x.T).reshape(_F * _NP)

    def splat(v):
        return jnp.broadcast_to(v.reshape(-1, 1), (v.size, _L))

    params = jnp.concatenate(
        [splat(W1.reshape(-1)), splat(W2.reshape(-1)), splat(Wlin.reshape(-1)),
         splat(b1), splat(b2), splat(blin)], axis=0).reshape(_NPAR * _L)

    ar = jnp.arange(_TBL, dtype=jnp.float32)
    tbl = jnp.where(ar > 0, ar ** -0.5, 0.0)

    out = _run(edge_index.astype(jnp.int32), xcm, params, tbl)
    return out[:_N].reshape(_N, 1)
